# baseline (speedup 1.0000x reference)
import jax
import jax.numpy as jnp
from jax import lax
from jax.experimental import pallas as pl
from jax.experimental.pallas import tpu as pltpu

_BM = 1024
_NT = (((1,), (1,)), ((), ()))


def _body(L_ref, x_ref, w_ref, b_ref, o_ref, zt_ref, rt_ref):
    i = pl.program_id(0)
    d = x_ref.shape[1]

    @pl.when(i == 0)
    def _():
        w1 = w_ref[:, :d]
        w2 = w_ref[:, d:]
        zt_ref[...] = lax.dot_general(
            w2, x_ref[...], _NT, preferred_element_type=jnp.float32
        ).astype(jnp.bfloat16)
        rt_ref[...] = (
            lax.dot_general(
                w1, x_ref[...], _NT, preferred_element_type=jnp.float32
            )
            + b_ref[...]
        )

    n = x_ref.shape[0]
    h = n // 2
    acc = lax.dot_general(
        zt_ref[:, :h],
        L_ref[:, :h].astype(jnp.bfloat16),
        _NT,
        preferred_element_type=jnp.float32,
    ) + lax.dot_general(
        zt_ref[:, h:],
        L_ref[:, h:].astype(jnp.bfloat16),
        _NT,
        preferred_element_type=jnp.float32,
    )
    o_ref[...] = (acc + rt_ref[:, pl.ds(i * _BM, _BM)]).T


@jax.jit
def kernel(L, x, W, b):
    n, d = x.shape
    out = W.shape[0]
    b2 = b.reshape(out, 1)

    return pl.pallas_call(
        _body,
        grid=(n // _BM,),
        in_specs=[
            pl.BlockSpec((_BM, n), lambda i: (i, 0)),
            pl.BlockSpec((n, d), lambda i: (0, 0)),
            pl.BlockSpec((out, 2 * d), lambda i: (0, 0)),
            pl.BlockSpec((out, 1), lambda i: (0, 0)),
        ],
        out_specs=pl.BlockSpec((_BM, out), lambda i: (i, 0)),
        out_shape=jax.ShapeDtypeStruct((n, out), jnp.float32),
        scratch_shapes=[
            pltpu.VMEM((out, n), jnp.bfloat16),
            pltpu.VMEM((out, n), jnp.float32),
        ],
    )(L, x, W, b2)


# speedup vs baseline: 1.0544x; 1.0544x over previous
import jax
import jax.numpy as jnp
from jax import lax
from jax.experimental import pallas as pl
from jax.experimental.pallas import tpu as pltpu

_BM = 1024


def _body(L_ref, xt_ref, w1_ref, w2_ref, b_ref, o_ref, zt_ref, rt_ref):
    i = pl.program_id(0)

    @pl.when(i == 0)
    def _():
        zt_ref[...] = jnp.dot(
            w2_ref[...], xt_ref[...], preferred_element_type=jnp.float32
        ).astype(jnp.bfloat16)
        rt_ref[...] = (
            jnp.dot(w1_ref[...], xt_ref[...], preferred_element_type=jnp.float32)
            + b_ref[...]
        )

    acc = lax.dot_general(
        zt_ref[...],
        L_ref[...].astype(jnp.bfloat16),
        ((( 1,), (1,)), ((), ())),
        preferred_element_type=jnp.float32,
    )
    o_ref[...] = (acc + rt_ref[:, pl.ds(i * _BM, _BM)]).T


@jax.jit
def kernel(L, x, W, b):
    n, d = x.shape
    out = W.shape[0]
    w1 = W[:, :d]
    w2 = W[:, d:]
    xt = x.T
    b2 = b.reshape(out, 1)

    return pl.pallas_call(
        _body,
        grid=(n // _BM,),
        in_specs=[
            pl.BlockSpec((_BM, n), lambda i: (i, 0)),
            pl.BlockSpec((d, n), lambda i: (0, 0)),
            pl.BlockSpec((out, d), lambda i: (0, 0)),
            pl.BlockSpec((out, d), lambda i: (0, 0)),
            pl.BlockSpec((out, 1), lambda i: (0, 0)),
        ],
        out_specs=pl.BlockSpec((_BM, out), lambda i: (i, 0)),
        out_shape=jax.ShapeDtypeStruct((n, out), jnp.float32),
        scratch_shapes=[
            pltpu.VMEM((out, n), jnp.bfloat16),
            pltpu.VMEM((out, n), jnp.float32),
        ],
    )(L, xt, w1, w2, b2)


# speedup vs baseline: 1.0760x; 1.0204x over previous
import jax
import jax.numpy as jnp
from jax import lax
from jax.experimental import pallas as pl
from jax.experimental.pallas import tpu as pltpu

_BM = 512


def _body(L_ref, xt_ref, w1_ref, w2_ref, b_ref, o_ref, zt_ref, rt_ref):
    i = pl.program_id(0)

    @pl.when(i == 0)
    def _():
        zt_ref[...] = jnp.dot(
            w2_ref[...], xt_ref[...], preferred_element_type=jnp.float32
        ).astype(jnp.bfloat16)
        rt_ref[...] = (
            jnp.dot(w1_ref[...], xt_ref[...], preferred_element_type=jnp.float32)
            + b_ref[...]
        )

    acc = lax.dot_general(
        zt_ref[...],
        L_ref[...].astype(jnp.bfloat16),
        ((( 1,), (1,)), ((), ())),
        preferred_element_type=jnp.float32,
    )
    o_ref[...] = (acc + rt_ref[:, pl.ds(i * _BM, _BM)]).T


@jax.jit
def kernel(L, x, W, b):
    n, d = x.shape
    out = W.shape[0]
    w1 = W[:, :d]
    w2 = W[:, d:]
    xt = x.T
    b2 = b.reshape(out, 1)

    return pl.pallas_call(
        _body,
        grid=(n // _BM,),
        in_specs=[
            pl.BlockSpec((_BM, n), lambda i: (i, 0)),
            pl.BlockSpec((d, n), lambda i: (0, 0)),
            pl.BlockSpec((out, d), lambda i: (0, 0)),
            pl.BlockSpec((out, d), lambda i: (0, 0)),
            pl.BlockSpec((out, 1), lambda i: (0, 0)),
        ],
        out_specs=pl.BlockSpec((_BM, out), lambda i: (i, 0)),
        out_shape=jax.ShapeDtypeStruct((n, out), jnp.float32),
        scratch_shapes=[
            pltpu.VMEM((out, n), jnp.bfloat16),
            pltpu.VMEM((out, n), jnp.float32),
        ],
    )(L, xt, w1, w2, b2)


# speedup vs baseline: 1.0929x; 1.0158x over previous
import jax
import jax.numpy as jnp
from jax import lax
from jax.experimental import pallas as pl
from jax.experimental.pallas import tpu as pltpu

_BM = 1024
_NK = 2
_NT = (((1,), (1,)), ((), ()))


def _body(L_ref, xt_ref, w1_ref, w2_ref, b_ref, o_ref, zt_ref, rt_ref, acc_ref):
    i = pl.program_id(0)
    j = pl.program_id(1)
    n = xt_ref.shape[1]
    kc = n // _NK

    @pl.when((i == 0) & (j == 0))
    def _():
        zt_ref[...] = jnp.dot(
            w2_ref[...], xt_ref[...], preferred_element_type=jnp.float32
        ).astype(jnp.bfloat16)
        rt_ref[...] = (
            jnp.dot(w1_ref[...], xt_ref[...], preferred_element_type=jnp.float32)
            + b_ref[...]
        )

    part = lax.dot_general(
        zt_ref[:, pl.ds(j * kc, kc)],
        L_ref[...].astype(jnp.bfloat16),
        _NT,
        preferred_element_type=jnp.float32,
    )

    @pl.when(j == 0)
    def _():
        acc_ref[...] = part

    @pl.when(j == _NK - 1)
    def _():
        o_ref[...] = (acc_ref[...] + part + rt_ref[:, pl.ds(i * _BM, _BM)]).T


@jax.jit
def kernel(L, x, W, b):
    n, d = x.shape
    out = W.shape[0]
    w1 = W[:, :d]
    w2 = W[:, d:]
    xt = x.T
    b2 = b.reshape(out, 1)
    kc = n // _NK

    return pl.pallas_call(
        _body,
        grid=(n // _BM, _NK),
        in_specs=[
            pl.BlockSpec((_BM, kc), lambda i, j: (i, j)),
            pl.BlockSpec((d, n), lambda i, j: (0, 0)),
            pl.BlockSpec((out, d), lambda i, j: (0, 0)),
            pl.BlockSpec((out, d), lambda i, j: (0, 0)),
            pl.BlockSpec((out, 1), lambda i, j: (0, 0)),
        ],
        out_specs=pl.BlockSpec((_BM, out), lambda i, j: (i, 0)),
        out_shape=jax.ShapeDtypeStruct((n, out), jnp.float32),
        scratch_shapes=[
            pltpu.VMEM((out, n), jnp.bfloat16),
            pltpu.VMEM((out, n), jnp.float32),
            pltpu.VMEM((out, _BM), jnp.float32),
        ],
    )(L, xt, w1, w2, b2)
